# Initial kernel scaffold; baseline (speedup 1.0000x reference)
import jax
import jax.numpy as jnp
from jax.experimental import pallas as pl


def kernel(user_id, timestamp, user_rating, user_occupation_label, raw_user_age, user_gender, occ_tokens, user_table, ts_table, occ_table):
    raise NotImplementedError("write your pallas kernel here")


# speedup vs baseline: 3.4538x; 3.4538x over previous
import jax
import jax.numpy as jnp
from jax import lax
from jax.experimental import pallas as pl
from jax.experimental.pallas import tpu as pltpu
from jax.experimental.pallas import tpu_sc as plsc

_NUM_BUCKETS = 1000
_EMBED_DIM = 32
_BATCH = 16384
_TOK_LEN = 20
_NORM_MEAN = 0.5
_NORM_STD = 0.2887
_OUT_D = 101

_NC = 2
_NS = 16
_NW = _NC * _NS
_ROWS_PER_W = _BATCH // _NW
_CHUNK = 128
_NCHUNK = _ROWS_PER_W // _CHUNK

_ZERO_ROW = 1002


def _body(uid_hbm, t_hbm, rate_hbm, occl_hbm, age_hbm, gen_hbm, tok_hbm,
          utab_hbm, tstab_hbm, occtab_hbm, bnd_hbm, out_hbm,
          occ_v, ts_v, bnd_v, uid_v, urows_v, t_v, rate_v, occl_v, age_v,
          gen_v, tok_v, out_v):
    wid = lax.axis_index("s") * _NC + lax.axis_index("c")
    base0 = wid * _ROWS_PER_W

    pltpu.sync_copy(occtab_hbm, occ_v.at[pl.ds(0, _NUM_BUCKETS + 2)])
    pltpu.sync_copy(tstab_hbm, ts_v)
    pltpu.sync_copy(bnd_hbm, bnd_v)
    zeros16 = jnp.zeros((16,), jnp.float32)
    occ_v[_ZERO_ROW, pl.ds(0, 16)] = zeros16
    occ_v[_ZERO_ROW, pl.ds(16, 16)] = zeros16

    iot = lax.iota(jnp.int32, 16)

    def chunk_body(ci, carry):
        base = base0 + ci * _CHUNK
        pltpu.sync_copy(uid_hbm.at[pl.ds(base, _CHUNK)], uid_v)
        pltpu.sync_copy(t_hbm.at[pl.ds(base, _CHUNK)], t_v)
        pltpu.sync_copy(rate_hbm.at[pl.ds(base, _CHUNK)], rate_v)
        pltpu.sync_copy(occl_hbm.at[pl.ds(base, _CHUNK)], occl_v)
        pltpu.sync_copy(age_hbm.at[pl.ds(base, _CHUNK)], age_v)
        pltpu.sync_copy(gen_hbm.at[pl.ds(base, _CHUNK)], gen_v)
        pltpu.sync_copy(tok_hbm.at[pl.ds(base, _CHUNK)], tok_v)
        pltpu.sync_copy(utab_hbm.at[uid_v], urows_v)

        def group_body(g, c2):
            r0 = g * 16
            rowvec = r0 + iot

            t = t_v[pl.ds(r0, 16)]
            k0 = jnp.clip((t * float(_NUM_BUCKETS - 1)).astype(jnp.int32) + 1,
                          1, _NUM_BUCKETS)
            b_lo = plsc.load_gather(bnd_v, [k0 - 1])
            b_hi = plsc.load_gather(bnd_v, [k0])
            idx = (k0 - (t < b_lo).astype(jnp.int32)
                   + (t >= b_hi).astype(jnp.int32))
            idx = jnp.clip(idx, 0, _NUM_BUCKETS + 1)

            nt = (t - _NORM_MEAN) / _NORM_STD
            rate = rate_v[pl.ds(r0, 16)]
            occl = occl_v[pl.ds(r0, 16)].astype(jnp.float32)
            age = age_v[pl.ds(r0, 16)]
            gen = gen_v[pl.ds(r0, 16)].astype(jnp.float32)
            plsc.store_scatter(out_v, [rowvec, jnp.full((16,), 64, jnp.int32)], nt)
            plsc.store_scatter(out_v, [rowvec, jnp.full((16,), 65, jnp.int32)], rate)
            plsc.store_scatter(out_v, [rowvec, jnp.full((16,), 66, jnp.int32)], occl)
            plsc.store_scatter(out_v, [rowvec, jnp.full((16,), 67, jnp.int32)], age)
            plsc.store_scatter(out_v, [rowvec, jnp.full((16,), 68, jnp.int32)], gen)

            toks = []
            n0 = jnp.zeros((16,), jnp.int32)
            for l in range(_TOK_LEN):
                tk = plsc.load_gather(tok_v, [rowvec, jnp.full((16,), l, jnp.int32)])
                z = tk == 0
                n0 = n0 + z.astype(jnp.int32)
                toks.append(jnp.where(z, _ZERO_ROW, tk))
            cnt = jnp.maximum(jnp.float32(_TOK_LEN) - n0.astype(jnp.float32), 1.0)
            inv = 1.0 / cnt

            for d in range(_EMBED_DIM):
                dv = jnp.full((16,), d, jnp.int32)
                uvec = plsc.load_gather(urows_v, [rowvec, dv])
                plsc.store_scatter(out_v, [rowvec, dv], uvec)
                tvec = plsc.load_gather(ts_v, [idx, dv])
                plsc.store_scatter(out_v, [rowvec, dv + 32], tvec)
                acc = plsc.load_gather(occ_v, [toks[0], dv])
                for l in range(1, _TOK_LEN):
                    acc = acc + plsc.load_gather(occ_v, [toks[l], dv])
                plsc.store_scatter(out_v, [rowvec, dv + 69], acc * inv)
            return c2

        lax.fori_loop(0, _CHUNK // 16, group_body, 0)
        pltpu.sync_copy(out_v, out_hbm.at[pl.ds(base, _CHUNK)])
        return carry

    lax.fori_loop(0, _NCHUNK, chunk_body, 0)


_sc_call = pl.kernel(
    _body,
    out_type=jax.ShapeDtypeStruct((_BATCH, _OUT_D), jnp.float32),
    mesh=plsc.VectorSubcoreMesh(core_axis_name="c", subcore_axis_name="s",
                                num_cores=_NC, num_subcores=_NS),
    scratch_types=[
        pltpu.VMEM((_NUM_BUCKETS + 3, _EMBED_DIM), jnp.float32),
        pltpu.VMEM((_NUM_BUCKETS + 2, _EMBED_DIM), jnp.float32),
        pltpu.VMEM((_NUM_BUCKETS + 8, ), jnp.float32),
        pltpu.VMEM((_CHUNK,), jnp.int32),
        pltpu.VMEM((_CHUNK, _EMBED_DIM), jnp.float32),
        pltpu.VMEM((_CHUNK,), jnp.float32),
        pltpu.VMEM((_CHUNK,), jnp.float32),
        pltpu.VMEM((_CHUNK,), jnp.int32),
        pltpu.VMEM((_CHUNK,), jnp.float32),
        pltpu.VMEM((_CHUNK,), jnp.int32),
        pltpu.VMEM((_CHUNK, _TOK_LEN), jnp.int32),
        pltpu.VMEM((_CHUNK, _OUT_D), jnp.float32),
    ],
    compiler_params=pltpu.CompilerParams(needs_layout_passes=False,
                                         use_tc_tiling_on_sc=False),
)


@jax.jit
def kernel(user_id, timestamp, user_rating, user_occupation_label,
           raw_user_age, user_gender, occ_tokens, user_table, ts_table,
           occ_table):
    boundaries = jnp.linspace(0.0, 1.0, _NUM_BUCKETS).astype(jnp.float32)
    bnd = jnp.concatenate([boundaries, jnp.full((8,), 2.0, jnp.float32)])
    return _sc_call(user_id, timestamp, user_rating, user_occupation_label,
                    raw_user_age, user_gender, occ_tokens, user_table,
                    ts_table, occ_table, bnd)


# speedup vs baseline: 3.4814x; 1.0080x over previous
import jax
import jax.numpy as jnp
from jax import lax
from jax.experimental import pallas as pl
from jax.experimental.pallas import tpu as pltpu
from jax.experimental.pallas import tpu_sc as plsc

_NUM_BUCKETS = 1000
_EMBED_DIM = 32
_BATCH = 16384
_TOK_LEN = 20
_NORM_MEAN = 0.5
_NORM_STD = 0.2887
_OUT_D = 101

_NC = 2
_NS = 16
_NW = _NC * _NS
_ROWS_PER_W = _BATCH // _NW
_CHUNK = 128
_NCHUNK = _ROWS_PER_W // _CHUNK
_NGROUP = _CHUNK // 16

_ZERO_ROW = 1002


def _body(uid_hbm, t_hbm, rate_hbm, occl_hbm, age_hbm, gen_hbm, tok_hbm,
          utab_hbm, tstab_hbm, occtab_hbm, bnd_hbm, out_hbm,
          occ_v, ts_v, bnd_v, uid_v, uid4_v, urows_v, t_v, rate_v, occl_v,
          age_v, gen_v, tok_v, out_v):
    wid = lax.axis_index("s") * _NC + lax.axis_index("c")
    base0 = wid * _ROWS_PER_W

    pltpu.sync_copy(occtab_hbm, occ_v.at[pl.ds(0, (_NUM_BUCKETS + 2) * _EMBED_DIM)])
    pltpu.sync_copy(tstab_hbm, ts_v)
    pltpu.sync_copy(bnd_hbm, bnd_v)
    pltpu.sync_copy(uid_hbm.at[pl.ds(base0, _ROWS_PER_W)], uid_v)
    pltpu.sync_copy(t_hbm.at[pl.ds(base0, _ROWS_PER_W)], t_v)
    pltpu.sync_copy(rate_hbm.at[pl.ds(base0, _ROWS_PER_W)], rate_v)
    pltpu.sync_copy(occl_hbm.at[pl.ds(base0, _ROWS_PER_W)], occl_v)
    pltpu.sync_copy(age_hbm.at[pl.ds(base0, _ROWS_PER_W)], age_v)
    pltpu.sync_copy(gen_hbm.at[pl.ds(base0, _ROWS_PER_W)], gen_v)
    pltpu.sync_copy(tok_hbm.at[pl.ds(base0 * _TOK_LEN, _ROWS_PER_W * _TOK_LEN)],
                    tok_v)
    zeros16 = jnp.zeros((16,), jnp.float32)
    occ_v[pl.ds(_ZERO_ROW * _EMBED_DIM, 16)] = zeros16
    occ_v[pl.ds(_ZERO_ROW * _EMBED_DIM + 16, 16)] = zeros16

    iot = lax.iota(jnp.int32, 16)
    iot101 = iot * _OUT_D
    iot20 = iot * _TOK_LEN

    for j in range(_ROWS_PER_W // 16):
        u = uid_v[pl.ds(j * 16, 16)]
        uid4_v[j // _NGROUP, pl.ds((j % _NGROUP) * 16, 16)] = u >> 2

    def chunk_body(ci, carry):
        pltpu.sync_copy(utab_hbm.at[uid4_v.at[ci]], urows_v)

        def group_body(g, c2):
            r0 = g * 16
            w0 = ci * _CHUNK + r0
            rowvec = r0 + iot
            fi = r0 * _OUT_D + iot101

            t = t_v[pl.ds(w0, 16)]
            k0 = jnp.clip((t * float(_NUM_BUCKETS - 1)).astype(jnp.int32) + 1,
                          1, _NUM_BUCKETS)
            b_lo = plsc.load_gather(bnd_v, [k0 - 1])
            b_hi = plsc.load_gather(bnd_v, [k0])
            idx = (k0 - (t < b_lo).astype(jnp.int32)
                   + (t >= b_hi).astype(jnp.int32))
            idx32 = jnp.clip(idx, 0, _NUM_BUCKETS + 1) * _EMBED_DIM

            nt = (t - _NORM_MEAN) / _NORM_STD
            rate = rate_v[pl.ds(w0, 16)]
            occl = occl_v[pl.ds(w0, 16)].astype(jnp.float32)
            age = age_v[pl.ds(w0, 16)]
            gen = gen_v[pl.ds(w0, 16)].astype(jnp.float32)
            plsc.store_scatter(out_v, [fi + 64], nt)
            plsc.store_scatter(out_v, [fi + 65], rate)
            plsc.store_scatter(out_v, [fi + 66], occl)
            plsc.store_scatter(out_v, [fi + 67], age)
            plsc.store_scatter(out_v, [fi + 68], gen)

            uid = uid_v[pl.ds(w0, 16)]
            ubase = (uid & 3) * _EMBED_DIM

            tokbase = w0 * _TOK_LEN + iot20
            tok32 = []
            n0 = jnp.zeros((16,), jnp.int32)
            for l in range(_TOK_LEN):
                tk = plsc.load_gather(tok_v, [tokbase + l])
                z = tk == 0
                n0 = n0 + z.astype(jnp.int32)
                tok32.append(jnp.where(z, _ZERO_ROW, tk) * _EMBED_DIM)
            cnt = jnp.maximum(jnp.float32(_TOK_LEN) - n0.astype(jnp.float32), 1.0)
            inv = 1.0 / cnt

            for d in range(_EMBED_DIM):
                uvec = plsc.load_gather(urows_v, [rowvec, ubase + d])
                plsc.store_scatter(out_v, [fi + d], uvec)
                tvec = plsc.load_gather(ts_v, [idx32 + d])
                plsc.store_scatter(out_v, [fi + (32 + d)], tvec)
                acc = plsc.load_gather(occ_v, [tok32[0] + d])
                for l in range(1, _TOK_LEN):
                    acc = acc + plsc.load_gather(occ_v, [tok32[l] + d])
                plsc.store_scatter(out_v, [fi + (69 + d)], acc * inv)
            return c2

        lax.fori_loop(0, _NGROUP, group_body, 0)
        pltpu.sync_copy(out_v,
                        out_hbm.at[pl.ds((base0 + ci * _CHUNK) * _OUT_D,
                                         _CHUNK * _OUT_D)])
        return carry

    lax.fori_loop(0, _NCHUNK, chunk_body, 0)


_sc_call = pl.kernel(
    _body,
    out_type=jax.ShapeDtypeStruct((_BATCH * _OUT_D,), jnp.float32),
    mesh=plsc.VectorSubcoreMesh(core_axis_name="c", subcore_axis_name="s",
                                num_cores=_NC, num_subcores=_NS),
    scratch_types=[
        pltpu.VMEM(((_NUM_BUCKETS + 3) * _EMBED_DIM,), jnp.float32),
        pltpu.VMEM(((_NUM_BUCKETS + 2) * _EMBED_DIM,), jnp.float32),
        pltpu.VMEM((_NUM_BUCKETS + 8,), jnp.float32),
        pltpu.VMEM((_ROWS_PER_W,), jnp.int32),
        pltpu.VMEM((_NCHUNK, _CHUNK), jnp.int32),
        pltpu.VMEM((_CHUNK, 4 * _EMBED_DIM), jnp.float32),
        pltpu.VMEM((_ROWS_PER_W,), jnp.float32),
        pltpu.VMEM((_ROWS_PER_W,), jnp.float32),
        pltpu.VMEM((_ROWS_PER_W,), jnp.int32),
        pltpu.VMEM((_ROWS_PER_W,), jnp.float32),
        pltpu.VMEM((_ROWS_PER_W,), jnp.int32),
        pltpu.VMEM((_ROWS_PER_W * _TOK_LEN,), jnp.int32),
        pltpu.VMEM((_CHUNK * _OUT_D,), jnp.float32),
    ],
    compiler_params=pltpu.CompilerParams(needs_layout_passes=False,
                                         use_tc_tiling_on_sc=False),
)


@jax.jit
def kernel(user_id, timestamp, user_rating, user_occupation_label,
           raw_user_age, user_gender, occ_tokens, user_table, ts_table,
           occ_table):
    boundaries = jnp.linspace(0.0, 1.0, _NUM_BUCKETS).astype(jnp.float32)
    bnd = jnp.concatenate([boundaries, jnp.full((8,), 2.0, jnp.float32)])
    out = _sc_call(user_id, timestamp, user_rating, user_occupation_label,
                   raw_user_age, user_gender, occ_tokens.reshape(-1),
                   user_table.reshape(-1, 4 * _EMBED_DIM),
                   ts_table.reshape(-1), occ_table.reshape(-1), bnd)
    return out.reshape(_BATCH, _OUT_D)
